# Initial kernel scaffold; baseline (speedup 1.0000x reference)
import jax
import jax.numpy as jnp
from jax.experimental import pallas as pl


def kernel(x, edge_index, batch, W1, b1, W2, b2, W3, b3, FW1, Fb1, FW2, Fb2):
    raise NotImplementedError("write your pallas kernel here")


# speedup vs baseline: 9.5283x; 9.5283x over previous
import functools

import jax
import jax.numpy as jnp
from jax import lax
from jax.experimental import pallas as pl
from jax.experimental.pallas import tpu as pltpu
from jax.experimental.pallas import tpu_sc as plsc

NEG = 0.01
NC = 2
NS = 16
BN = 400
K = 128


def _sc_degree(col, n_nodes):
    e = col.shape[0]
    epc = e // NC
    ept = epc // NS
    nch = ept // K
    tail = ept - nch * K
    strip = (-(-n_nodes // NS) + 7) // 8 * 8
    mesh = plsc.VectorSubcoreMesh(core_axis_name="c", subcore_axis_name="s",
                                  num_cores=NC, num_subcores=NS)

    @functools.partial(
        pl.kernel,
        out_type=jax.ShapeDtypeStruct((NC * n_nodes, 128), jnp.float32),
        mesh=mesh,
        scratch_types=[
            pltpu.VMEM((K,), jnp.int32),
            pltpu.VMEM((tail,), jnp.int32),
            pltpu.VMEM((K, 128), jnp.float32),
            pltpu.MemorySpace.VMEM_SHARED((n_nodes, 128), jnp.float32),
        ],
    )
    def deg_kernel(col_hbm, ones_hbm, z_hbm, out_hbm, cidx, cidx_t, ones_v, acc):
        c = lax.axis_index("c")
        s = lax.axis_index("s")
        ro = jnp.minimum(s * strip, n_nodes - strip)
        pltpu.sync_copy(z_hbm.at[pl.ds(ro, strip)], acc.at[pl.ds(ro, strip)])
        pltpu.sync_copy(ones_hbm, ones_v)
        plsc.subcore_barrier()
        base0 = c * epc + s * ept

        def chunk(k, carry):
            b = base0 + k * K
            pltpu.sync_copy(col_hbm.at[pl.ds(b, K)], cidx)
            pltpu.sync_copy(ones_v, acc.at[cidx], add=True)
            return carry

        lax.fori_loop(0, nch, chunk, 0)
        if tail:
            b = base0 + nch * K
            pltpu.sync_copy(col_hbm.at[pl.ds(b, tail)], cidx_t)
            pltpu.sync_copy(ones_v.at[pl.ds(0, tail)], acc.at[cidx_t], add=True)
        plsc.subcore_barrier()
        pltpu.sync_copy(acc.at[pl.ds(ro, strip)],
                        out_hbm.at[pl.ds(c * n_nodes + ro, strip)])

    ones = jnp.ones((K, 128), jnp.float32)
    zeros = jnp.zeros((n_nodes, 128), jnp.float32)
    return deg_kernel(col, ones, zeros)


def _sc_aggregate(hp2, row, col, n_nodes):
    e = row.shape[0]
    ept = e // NS
    nch = ept // K
    tail = ept - nch * K
    strip = (-(-n_nodes // NS) + 7) // 8 * 8
    mesh = plsc.VectorSubcoreMesh(core_axis_name="c", subcore_axis_name="s",
                                  num_cores=NC, num_subcores=NS)

    @functools.partial(
        pl.kernel,
        out_type=jax.ShapeDtypeStruct((NC * n_nodes, 128), jnp.float32),
        mesh=mesh,
        scratch_types=[
            pltpu.VMEM((K,), jnp.int32),
            pltpu.VMEM((K,), jnp.int32),
            pltpu.VMEM((K, 128), jnp.float32),
            pltpu.VMEM((tail,), jnp.int32),
            pltpu.VMEM((tail,), jnp.int32),
            pltpu.VMEM((tail, 128), jnp.float32),
            pltpu.MemorySpace.VMEM_SHARED((n_nodes, 128), jnp.float32),
            pltpu.SemaphoreType.DMA,
        ],
    )
    def agg_kernel(row_hbm, col_hbm, h_hbm, z_hbm, out_hbm,
                   ridx, cidx, rows, ridx_t, cidx_t, rows_t, acc, sem):
        c = lax.axis_index("c")
        s = lax.axis_index("s")
        off = jnp.full((16,), c * n_nodes, jnp.int32)
        ro = jnp.minimum(s * strip, n_nodes - strip)
        pltpu.sync_copy(z_hbm.at[pl.ds(ro, strip)], acc.at[pl.ds(ro, strip)])
        plsc.subcore_barrier()
        base0 = s * ept

        def chunk(k, carry):
            b = base0 + k * K
            pltpu.sync_copy(row_hbm.at[pl.ds(b, K)], ridx)
            for j in range(K // 16):
                ridx[pl.ds(j * 16, 16)] = ridx[pl.ds(j * 16, 16)] + off
            pltpu.async_copy(h_hbm.at[ridx], rows, sem).wait()
            pltpu.sync_copy(col_hbm.at[pl.ds(b, K)], cidx)
            pltpu.sync_copy(rows, acc.at[cidx], add=True)
            return carry

        lax.fori_loop(0, nch, chunk, 0)
        if tail:
            b = base0 + nch * K
            pltpu.sync_copy(row_hbm.at[pl.ds(b, tail)], ridx_t)
            for j in range(tail // 16):
                ridx_t[pl.ds(j * 16, 16)] = ridx_t[pl.ds(j * 16, 16)] + off
            pltpu.async_copy(h_hbm.at[ridx_t], rows_t, sem).wait()
            pltpu.sync_copy(col_hbm.at[pl.ds(b, tail)], cidx_t)
            pltpu.sync_copy(rows_t, acc.at[cidx_t], add=True)
        plsc.subcore_barrier()
        pltpu.sync_copy(acc.at[pl.ds(ro, strip)],
                        out_hbm.at[pl.ds(c * n_nodes + ro, strip)])

    zeros = jnp.zeros((n_nodes, 128), jnp.float32)
    return agg_kernel(row, col, hp2, zeros)


def _mm1_body(x_ref, w_ref, dp_ref, hp_ref, dis_ref):
    dp = dp_ref[0, :, 0:1] + dp_ref[1, :, 0:1] + 1.0
    dis = lax.rsqrt(dp)
    h = jnp.dot(x_ref[...], w_ref[...], preferred_element_type=jnp.float32)
    hp = h * dis
    hp_ref[0] = hp[:, :128]
    hp_ref[1] = hp[:, 128:]
    dis_ref[...] = dis


def _tc_mm1(x, w1, degpad, n_nodes):
    f_in = x.shape[1]
    return pl.pallas_call(
        _mm1_body,
        grid=(n_nodes // BN,),
        in_specs=[
            pl.BlockSpec((BN, f_in), lambda i: (i, 0)),
            pl.BlockSpec((f_in, 256), lambda i: (0, 0)),
            pl.BlockSpec((2, BN, 128), lambda i: (0, i, 0)),
        ],
        out_specs=[
            pl.BlockSpec((2, BN, 128), lambda i: (0, i, 0)),
            pl.BlockSpec((BN, 1), lambda i: (i, 0)),
        ],
        out_shape=[
            jax.ShapeDtypeStruct((2, n_nodes, 128), jnp.float32),
            jax.ShapeDtypeStruct((n_nodes, 1), jnp.float32),
        ],
    )(x, w1, degpad)


def _actmm_body(s_ref, hp_ref, dis_ref, b_ref, w_ref, o_ref):
    dis = dis_ref[...]
    t = jnp.concatenate([s_ref[0] + hp_ref[0], s_ref[1] + hp_ref[1]], axis=1)
    pre = t * dis + b_ref[...]
    a = jnp.where(pre >= 0, pre, NEG * pre)
    h = jnp.dot(a, w_ref[...], preferred_element_type=jnp.float32)
    hp = h * dis
    o_ref[0] = hp[:, :128]
    o_ref[1] = hp[:, 128:]


def _tc_actmm(s3, hp, dis, b, w, n_nodes):
    return pl.pallas_call(
        _actmm_body,
        grid=(n_nodes // BN,),
        in_specs=[
            pl.BlockSpec((2, BN, 128), lambda i: (0, i, 0)),
            pl.BlockSpec((2, BN, 128), lambda i: (0, i, 0)),
            pl.BlockSpec((BN, 1), lambda i: (i, 0)),
            pl.BlockSpec((1, 256), lambda i: (0, 0)),
            pl.BlockSpec((256, 256), lambda i: (0, 0)),
        ],
        out_specs=pl.BlockSpec((2, BN, 128), lambda i: (0, i, 0)),
        out_shape=jax.ShapeDtypeStruct((2, n_nodes, 128), jnp.float32),
    )(s3, hp, dis, b, w)


def _final_body(s_ref, hp_ref, dis_ref, b_ref, bt_ref,
                fw1_ref, fb1_ref, fw2_ref, fb2_ref, o_ref, sums, cnts):
    i = pl.program_id(0)

    @pl.when(i == 0)
    def _init():
        sums[...] = jnp.zeros_like(sums)
        cnts[...] = jnp.zeros_like(cnts)

    dis = dis_ref[...]
    t = jnp.concatenate([s_ref[0] + hp_ref[0], s_ref[1] + hp_ref[1]], axis=1)
    out3 = t * dis + b_ref[...]
    gi = lax.broadcasted_iota(jnp.int32, (BN, 64), 1)
    oh = (gi == bt_ref[...]).astype(jnp.float32)
    sums[...] += lax.dot_general(oh, out3, (((0,), (0,)), ((), ())),
                                 preferred_element_type=jnp.float32)
    cnts[...] += lax.dot_general(oh, jnp.ones((BN, 128), jnp.float32),
                                 (((0,), (0,)), ((), ())),
                                 preferred_element_type=jnp.float32)

    @pl.when(i == pl.num_programs(0) - 1)
    def _fin():
        cnt = jnp.concatenate([cnts[...], cnts[...]], axis=1)
        emb = sums[...] / jnp.maximum(cnt, 1.0)
        e1 = jnp.dot(emb, fw1_ref[...], preferred_element_type=jnp.float32)
        e1 = e1 + fb1_ref[...]
        e1 = jnp.where(e1 >= 0, e1, NEG * e1)
        out = jnp.dot(e1, fw2_ref[...], preferred_element_type=jnp.float32)
        o_ref[...] = out + fb2_ref[...]


def _tc_final(s3, hp, dis, b, batch2, fw1, fb1, fw2, fb2, n_nodes, n_cls):
    return pl.pallas_call(
        _final_body,
        grid=(n_nodes // BN,),
        in_specs=[
            pl.BlockSpec((2, BN, 128), lambda i: (0, i, 0)),
            pl.BlockSpec((2, BN, 128), lambda i: (0, i, 0)),
            pl.BlockSpec((BN, 1), lambda i: (i, 0)),
            pl.BlockSpec((1, 256), lambda i: (0, 0)),
            pl.BlockSpec((BN, 1), lambda i: (i, 0)),
            pl.BlockSpec((256, 256), lambda i: (0, 0)),
            pl.BlockSpec((1, 256), lambda i: (0, 0)),
            pl.BlockSpec((256, n_cls), lambda i: (0, 0)),
            pl.BlockSpec((1, n_cls), lambda i: (0, 0)),
        ],
        out_specs=pl.BlockSpec((64, n_cls), lambda i: (0, 0)),
        out_shape=jax.ShapeDtypeStruct((64, n_cls), jnp.float32),
        scratch_shapes=[
            pltpu.VMEM((64, 256), jnp.float32),
            pltpu.VMEM((64, 128), jnp.float32),
        ],
        compiler_params=pltpu.CompilerParams(
            dimension_semantics=("arbitrary",)),
    )(s3, hp, dis, b, batch2, fw1, fb1, fw2, fb2)


def kernel(x, edge_index, batch, W1, b1, W2, b2, W3, b3, FW1, Fb1, FW2, Fb2):
    n = x.shape[0]
    row = edge_index[0]
    col = edge_index[1]
    batch2 = batch.reshape(n, 1)
    b1r = b1.reshape(1, -1)
    b2r = b2.reshape(1, -1)
    b3r = b3.reshape(1, -1)
    fb1r = Fb1.reshape(1, -1)
    fb2r = Fb2.reshape(1, -1)

    degpad = _sc_degree(col, n).reshape(2, n, 128)
    hp1, dis = _tc_mm1(x, W1, degpad, n)
    s1 = _sc_aggregate(hp1.reshape(2 * n, 128), row, col, n).reshape(2, n, 128)
    hp2 = _tc_actmm(s1, hp1, dis, b1r, W2, n)
    s2 = _sc_aggregate(hp2.reshape(2 * n, 128), row, col, n).reshape(2, n, 128)
    hp3 = _tc_actmm(s2, hp2, dis, b2r, W3, n)
    s3 = _sc_aggregate(hp3.reshape(2 * n, 128), row, col, n).reshape(2, n, 128)
    return _tc_final(s3, hp3, dis, b3r, batch2, FW1, fb1r, FW2, fb2r,
                     n, FW2.shape[1])


# speedup vs baseline: 15.0755x; 1.5822x over previous
import functools

import jax
import jax.numpy as jnp
from jax import lax
from jax.experimental import pallas as pl
from jax.experimental.pallas import tpu as pltpu
from jax.experimental.pallas import tpu_sc as plsc

NEG = 0.01
NC = 2
NS = 16
BN = 400
K = 128


def _sc_degree(col, n_nodes):
    e = col.shape[0]
    epc = e // NC
    ept = epc // NS
    nch = ept // K
    tail = ept - nch * K
    strip = (-(-n_nodes // NS) + 7) // 8 * 8
    mesh = plsc.VectorSubcoreMesh(core_axis_name="c", subcore_axis_name="s",
                                  num_cores=NC, num_subcores=NS)

    @functools.partial(
        pl.kernel,
        out_type=jax.ShapeDtypeStruct((NC * n_nodes, 128), jnp.float32),
        mesh=mesh,
        scratch_types=[
            pltpu.VMEM((K,), jnp.int32),
            pltpu.VMEM((tail,), jnp.int32),
            pltpu.VMEM((K, 128), jnp.float32),
            pltpu.MemorySpace.VMEM_SHARED((n_nodes, 128), jnp.float32),
        ],
    )
    def deg_kernel(col_hbm, ones_hbm, z_hbm, out_hbm, cidx, cidx_t, ones_v, acc):
        c = lax.axis_index("c")
        s = lax.axis_index("s")
        ro = jnp.minimum(s * strip, n_nodes - strip)
        pltpu.sync_copy(z_hbm.at[pl.ds(ro, strip)], acc.at[pl.ds(ro, strip)])
        pltpu.sync_copy(ones_hbm, ones_v)
        plsc.subcore_barrier()
        base0 = c * epc + s * ept

        def chunk(k, carry):
            b = base0 + k * K
            pltpu.sync_copy(col_hbm.at[pl.ds(b, K)], cidx)
            pltpu.sync_copy(ones_v, acc.at[cidx], add=True)
            return carry

        lax.fori_loop(0, nch, chunk, 0)
        if tail:
            b = base0 + nch * K
            pltpu.sync_copy(col_hbm.at[pl.ds(b, tail)], cidx_t)
            pltpu.sync_copy(ones_v.at[pl.ds(0, tail)], acc.at[cidx_t], add=True)
        plsc.subcore_barrier()
        pltpu.sync_copy(acc.at[pl.ds(ro, strip)],
                        out_hbm.at[pl.ds(c * n_nodes + ro, strip)])

    ones = jnp.ones((K, 128), jnp.float32)
    zeros = jnp.zeros((n_nodes, 128), jnp.float32)
    return deg_kernel(col, ones, zeros)


def _sc_aggregate(hp2, row, col, n_nodes):
    e = row.shape[0]
    ept = e // NS
    nch = ept // K
    tail = ept - nch * K
    strip = (-(-n_nodes // NS) + 7) // 8 * 8
    mesh = plsc.VectorSubcoreMesh(core_axis_name="c", subcore_axis_name="s",
                                  num_cores=NC, num_subcores=NS)

    assert nch % 2 == 0 and nch >= 4

    @functools.partial(
        pl.kernel,
        out_type=jax.ShapeDtypeStruct((NC * n_nodes, 128), jnp.float32),
        mesh=mesh,
        scratch_types=[
            pltpu.VMEM((K,), jnp.int32), pltpu.VMEM((K,), jnp.int32),
            pltpu.VMEM((K,), jnp.int32), pltpu.VMEM((K,), jnp.int32),
            pltpu.VMEM((K, 128), jnp.float32), pltpu.VMEM((K, 128), jnp.float32),
            pltpu.VMEM((tail,), jnp.int32),
            pltpu.VMEM((tail,), jnp.int32),
            pltpu.VMEM((tail, 128), jnp.float32),
            pltpu.MemorySpace.VMEM_SHARED((n_nodes, 128), jnp.float32),
            pltpu.SemaphoreType.DMA, pltpu.SemaphoreType.DMA,
            pltpu.SemaphoreType.DMA, pltpu.SemaphoreType.DMA,
            pltpu.SemaphoreType.DMA, pltpu.SemaphoreType.DMA,
            pltpu.SemaphoreType.DMA, pltpu.SemaphoreType.DMA,
            pltpu.SemaphoreType.DMA,
        ],
    )
    def agg_kernel(row_hbm, col_hbm, h_hbm, z_hbm, out_hbm,
                   ridx0, ridx1, cidx0, cidx1, rows0, rows1,
                   ridx_t, cidx_t, rows_t, acc,
                   rs0, rs1, cs0, cs1, gs0, gs1, ss0, ss1, sem_t):
        c = lax.axis_index("c")
        s = lax.axis_index("s")
        off = jnp.full((16,), c * n_nodes, jnp.int32)
        ro = jnp.minimum(s * strip, n_nodes - strip)
        pltpu.sync_copy(z_hbm.at[pl.ds(ro, strip)], acc.at[pl.ds(ro, strip)])
        plsc.subcore_barrier()
        base0 = s * ept
        ridx = (ridx0, ridx1)
        cidx = (cidx0, cidx1)
        rows = (rows0, rows1)
        rs = (rs0, rs1)
        cs = (cs0, cs1)
        gs = (gs0, gs1)
        ss = (ss0, ss1)

        def start_row_idx(k, b):
            pltpu.async_copy(row_hbm.at[pl.ds(base0 + k * K, K)], ridx[b], rs[b])

        def start_col_idx(k, b):
            pltpu.async_copy(col_hbm.at[pl.ds(base0 + k * K, K)], cidx[b], cs[b])

        def start_gather(b):
            pltpu.make_async_copy(row_hbm.at[pl.ds(base0, K)], ridx[b], rs[b]).wait()
            for j in range(K // 16):
                ridx[b][pl.ds(j * 16, 16)] = ridx[b][pl.ds(j * 16, 16)] + off
            pltpu.async_copy(h_hbm.at[ridx[b]], rows[b], gs[b])

        def start_scatter(b):
            pltpu.make_async_copy(h_hbm.at[ridx[b]], rows[b], gs[b]).wait()
            pltpu.make_async_copy(col_hbm.at[pl.ds(base0, K)], cidx[b], cs[b]).wait()
            pltpu.async_copy(rows[b], acc.at[cidx[b]], ss[b], add=True)

        def wait_scatter(b):
            pltpu.make_async_copy(rows[b], acc.at[cidx[b]], ss[b]).wait()

        start_row_idx(0, 0)
        start_col_idx(0, 0)
        start_row_idx(1, 1)
        start_col_idx(1, 1)
        start_gather(0)
        start_gather(1)

        def pair(k2, carry):
            k = 2 * k2
            start_scatter(0)
            start_row_idx(k + 2, 0)
            start_scatter(1)
            start_row_idx(k + 3, 1)
            wait_scatter(0)
            start_col_idx(k + 2, 0)
            start_gather(0)
            wait_scatter(1)
            start_col_idx(k + 3, 1)
            start_gather(1)
            return carry

        lax.fori_loop(0, nch // 2 - 1, pair, 0)
        start_scatter(0)
        start_scatter(1)
        if tail:
            b = base0 + nch * K
            pltpu.sync_copy(row_hbm.at[pl.ds(b, tail)], ridx_t)
            for j in range(tail // 16):
                ridx_t[pl.ds(j * 16, 16)] = ridx_t[pl.ds(j * 16, 16)] + off
            pltpu.async_copy(h_hbm.at[ridx_t], rows_t, sem_t).wait()
            pltpu.sync_copy(col_hbm.at[pl.ds(b, tail)], cidx_t)
            pltpu.async_copy(rows_t, acc.at[cidx_t], sem_t, add=True)
            pltpu.make_async_copy(rows_t, acc.at[cidx_t], sem_t).wait()
        wait_scatter(0)
        wait_scatter(1)
        plsc.subcore_barrier()
        pltpu.sync_copy(acc.at[pl.ds(ro, strip)],
                        out_hbm.at[pl.ds(c * n_nodes + ro, strip)])

    zeros = jnp.zeros((n_nodes, 128), jnp.float32)
    return agg_kernel(row, col, hp2, zeros)


def _mm1_body(x_ref, w_ref, dp_ref, hp_ref, dis_ref):
    dp = dp_ref[0, :, 0:1] + dp_ref[1, :, 0:1] + 1.0
    dis = lax.rsqrt(dp)
    h = jnp.dot(x_ref[...], w_ref[...], preferred_element_type=jnp.float32)
    hp = h * dis
    hp_ref[0] = hp[:, :128]
    hp_ref[1] = hp[:, 128:]
    dis_ref[...] = dis


def _tc_mm1(x, w1, degpad, n_nodes):
    f_in = x.shape[1]
    return pl.pallas_call(
        _mm1_body,
        grid=(n_nodes // BN,),
        in_specs=[
            pl.BlockSpec((BN, f_in), lambda i: (i, 0)),
            pl.BlockSpec((f_in, 256), lambda i: (0, 0)),
            pl.BlockSpec((2, BN, 128), lambda i: (0, i, 0)),
        ],
        out_specs=[
            pl.BlockSpec((2, BN, 128), lambda i: (0, i, 0)),
            pl.BlockSpec((BN, 1), lambda i: (i, 0)),
        ],
        out_shape=[
            jax.ShapeDtypeStruct((2, n_nodes, 128), jnp.float32),
            jax.ShapeDtypeStruct((n_nodes, 1), jnp.float32),
        ],
    )(x, w1, degpad)


def _actmm_body(s_ref, hp_ref, dis_ref, b_ref, w_ref, o_ref):
    dis = dis_ref[...]
    t = jnp.concatenate([s_ref[0] + hp_ref[0], s_ref[1] + hp_ref[1]], axis=1)
    pre = t * dis + b_ref[...]
    a = jnp.where(pre >= 0, pre, NEG * pre)
    h = jnp.dot(a, w_ref[...], preferred_element_type=jnp.float32)
    hp = h * dis
    o_ref[0] = hp[:, :128]
    o_ref[1] = hp[:, 128:]


def _tc_actmm(s3, hp, dis, b, w, n_nodes):
    return pl.pallas_call(
        _actmm_body,
        grid=(n_nodes // BN,),
        in_specs=[
            pl.BlockSpec((2, BN, 128), lambda i: (0, i, 0)),
            pl.BlockSpec((2, BN, 128), lambda i: (0, i, 0)),
            pl.BlockSpec((BN, 1), lambda i: (i, 0)),
            pl.BlockSpec((1, 256), lambda i: (0, 0)),
            pl.BlockSpec((256, 256), lambda i: (0, 0)),
        ],
        out_specs=pl.BlockSpec((2, BN, 128), lambda i: (0, i, 0)),
        out_shape=jax.ShapeDtypeStruct((2, n_nodes, 128), jnp.float32),
    )(s3, hp, dis, b, w)


def _final_body(s_ref, hp_ref, dis_ref, b_ref, bt_ref,
                fw1_ref, fb1_ref, fw2_ref, fb2_ref, o_ref, sums, cnts):
    i = pl.program_id(0)

    @pl.when(i == 0)
    def _init():
        sums[...] = jnp.zeros_like(sums)
        cnts[...] = jnp.zeros_like(cnts)

    dis = dis_ref[...]
    t = jnp.concatenate([s_ref[0] + hp_ref[0], s_ref[1] + hp_ref[1]], axis=1)
    out3 = t * dis + b_ref[...]
    gi = lax.broadcasted_iota(jnp.int32, (BN, 64), 1)
    oh = (gi == bt_ref[...]).astype(jnp.float32)
    sums[...] += lax.dot_general(oh, out3, (((0,), (0,)), ((), ())),
                                 preferred_element_type=jnp.float32)
    cnts[...] += lax.dot_general(oh, jnp.ones((BN, 128), jnp.float32),
                                 (((0,), (0,)), ((), ())),
                                 preferred_element_type=jnp.float32)

    @pl.when(i == pl.num_programs(0) - 1)
    def _fin():
        cnt = jnp.concatenate([cnts[...], cnts[...]], axis=1)
        emb = sums[...] / jnp.maximum(cnt, 1.0)
        e1 = jnp.dot(emb, fw1_ref[...], preferred_element_type=jnp.float32)
        e1 = e1 + fb1_ref[...]
        e1 = jnp.where(e1 >= 0, e1, NEG * e1)
        out = jnp.dot(e1, fw2_ref[...], preferred_element_type=jnp.float32)
        o_ref[...] = out + fb2_ref[...]


def _tc_final(s3, hp, dis, b, batch2, fw1, fb1, fw2, fb2, n_nodes, n_cls):
    return pl.pallas_call(
        _final_body,
        grid=(n_nodes // BN,),
        in_specs=[
            pl.BlockSpec((2, BN, 128), lambda i: (0, i, 0)),
            pl.BlockSpec((2, BN, 128), lambda i: (0, i, 0)),
            pl.BlockSpec((BN, 1), lambda i: (i, 0)),
            pl.BlockSpec((1, 256), lambda i: (0, 0)),
            pl.BlockSpec((BN, 1), lambda i: (i, 0)),
            pl.BlockSpec((256, 256), lambda i: (0, 0)),
            pl.BlockSpec((1, 256), lambda i: (0, 0)),
            pl.BlockSpec((256, n_cls), lambda i: (0, 0)),
            pl.BlockSpec((1, n_cls), lambda i: (0, 0)),
        ],
        out_specs=pl.BlockSpec((64, n_cls), lambda i: (0, 0)),
        out_shape=jax.ShapeDtypeStruct((64, n_cls), jnp.float32),
        scratch_shapes=[
            pltpu.VMEM((64, 256), jnp.float32),
            pltpu.VMEM((64, 128), jnp.float32),
        ],
        compiler_params=pltpu.CompilerParams(
            dimension_semantics=("arbitrary",)),
    )(s3, hp, dis, b, batch2, fw1, fb1, fw2, fb2)


def kernel(x, edge_index, batch, W1, b1, W2, b2, W3, b3, FW1, Fb1, FW2, Fb2):
    n = x.shape[0]
    row = edge_index[0]
    col = edge_index[1]
    batch2 = batch.reshape(n, 1)
    b1r = b1.reshape(1, -1)
    b2r = b2.reshape(1, -1)
    b3r = b3.reshape(1, -1)
    fb1r = Fb1.reshape(1, -1)
    fb2r = Fb2.reshape(1, -1)

    degpad = _sc_degree(col, n).reshape(2, n, 128)
    hp1, dis = _tc_mm1(x, W1, degpad, n)
    s1 = _sc_aggregate(hp1.reshape(2 * n, 128), row, col, n).reshape(2, n, 128)
    hp2 = _tc_actmm(s1, hp1, dis, b1r, W2, n)
    s2 = _sc_aggregate(hp2.reshape(2 * n, 128), row, col, n).reshape(2, n, 128)
    hp3 = _tc_actmm(s2, hp2, dis, b2r, W3, n)
    s3 = _sc_aggregate(hp3.reshape(2 * n, 128), row, col, n).reshape(2, n, 128)
    return _tc_final(s3, hp3, dis, b3r, batch2, FW1, fb1r, FW2, fb2r,
                     n, FW2.shape[1])


# speedup vs baseline: 15.4951x; 1.0278x over previous
import functools

import jax
import jax.numpy as jnp
from jax import lax
from jax.experimental import pallas as pl
from jax.experimental.pallas import tpu as pltpu
from jax.experimental.pallas import tpu_sc as plsc

NEG = 0.01
NC = 2
NS = 16
BN = 400
K = 128


def _sc_degree(col, n_nodes):
    e = col.shape[0]
    epc = e // NC
    ept = epc // NS
    nch = ept // K
    tail = ept - nch * K
    strip = (-(-n_nodes // NS) + 7) // 8 * 8
    mesh = plsc.VectorSubcoreMesh(core_axis_name="c", subcore_axis_name="s",
                                  num_cores=NC, num_subcores=NS)

    assert nch % 2 == 0 and nch >= 4

    @functools.partial(
        pl.kernel,
        out_type=jax.ShapeDtypeStruct((NC * n_nodes, 128), jnp.float32),
        mesh=mesh,
        scratch_types=[
            pltpu.VMEM((K,), jnp.int32), pltpu.VMEM((K,), jnp.int32),
            pltpu.VMEM((tail,), jnp.int32),
            pltpu.VMEM((K, 128), jnp.float32),
            pltpu.MemorySpace.VMEM_SHARED((n_nodes, 128), jnp.float32),
            pltpu.SemaphoreType.DMA, pltpu.SemaphoreType.DMA,
            pltpu.SemaphoreType.DMA, pltpu.SemaphoreType.DMA,
            pltpu.SemaphoreType.DMA,
        ],
    )
    def deg_kernel(col_hbm, ones_hbm, z_hbm, out_hbm,
                   cidx0, cidx1, cidx_t, ones_v, acc,
                   cs0, cs1, ss0, ss1, sem_t):
        c = lax.axis_index("c")
        s = lax.axis_index("s")
        cidx = (cidx0, cidx1)
        cs = (cs0, cs1)
        ss = (ss0, ss1)
        ro = jnp.minimum(s * strip, n_nodes - strip)
        base0 = c * epc + s * ept

        def start_idx(k, b):
            pltpu.async_copy(col_hbm.at[pl.ds(base0 + k * K, K)], cidx[b], cs[b])

        def start_scatter(b):
            pltpu.make_async_copy(col_hbm.at[pl.ds(base0, K)], cidx[b], cs[b]).wait()
            pltpu.async_copy(ones_v, acc.at[cidx[b]], ss[b], add=True)

        def wait_scatter(b):
            pltpu.make_async_copy(ones_v, acc.at[cidx[b]], ss[b]).wait()

        start_idx(0, 0)
        start_idx(1, 1)
        pltpu.sync_copy(z_hbm.at[pl.ds(ro, strip)], acc.at[pl.ds(ro, strip)])
        pltpu.sync_copy(ones_hbm, ones_v)
        plsc.subcore_barrier()

        def pair(k2, carry):
            k = 2 * k2
            start_scatter(0)
            start_scatter(1)
            wait_scatter(0)
            start_idx(k + 2, 0)
            wait_scatter(1)
            start_idx(k + 3, 1)
            return carry

        lax.fori_loop(0, nch // 2 - 1, pair, 0)
        start_scatter(0)
        start_scatter(1)
        if tail:
            b = base0 + nch * K
            pltpu.sync_copy(col_hbm.at[pl.ds(b, tail)], cidx_t)
            pltpu.async_copy(ones_v.at[pl.ds(0, tail)], acc.at[cidx_t], sem_t,
                             add=True)
            pltpu.make_async_copy(ones_v.at[pl.ds(0, tail)], acc.at[cidx_t],
                                  sem_t).wait()
        wait_scatter(0)
        wait_scatter(1)
        plsc.subcore_barrier()
        pltpu.sync_copy(acc.at[pl.ds(ro, strip)],
                        out_hbm.at[pl.ds(c * n_nodes + ro, strip)])

    ones = jnp.ones((K, 128), jnp.float32)
    zeros = jnp.zeros((n_nodes, 128), jnp.float32)
    return deg_kernel(col, ones, zeros)


def _sc_aggregate(hp2, rowcat, col, n_nodes):
    e = rowcat.shape[0] // 2
    ept = e // NS
    nch = ept // K
    tail = ept - nch * K
    strip = (-(-n_nodes // NS) + 7) // 8 * 8
    mesh = plsc.VectorSubcoreMesh(core_axis_name="c", subcore_axis_name="s",
                                  num_cores=NC, num_subcores=NS)

    NB = 2
    assert nch % NB == 0 and nch >= 2 * NB

    @functools.partial(
        pl.kernel,
        out_type=jax.ShapeDtypeStruct((NC * n_nodes, 128), jnp.float32),
        mesh=mesh,
        scratch_types=(
            [pltpu.VMEM((K,), jnp.int32)] * NB
            + [pltpu.VMEM((K,), jnp.int32)] * NB
            + [pltpu.VMEM((K, 128), jnp.float32)] * NB
            + [pltpu.VMEM((tail,), jnp.int32),
               pltpu.VMEM((tail,), jnp.int32),
               pltpu.VMEM((tail, 128), jnp.float32),
               pltpu.MemorySpace.VMEM_SHARED((n_nodes, 128), jnp.float32)]
            + [pltpu.SemaphoreType.DMA] * (4 * NB + 1)
        ),
    )
    def agg_kernel(row_hbm, col_hbm, h_hbm, z_hbm, out_hbm, *bufs):
        ridx = bufs[0:NB]
        cidx = bufs[NB:2 * NB]
        rows = bufs[2 * NB:3 * NB]
        ridx_t, cidx_t, rows_t, acc = bufs[3 * NB:3 * NB + 4]
        sems = bufs[3 * NB + 4:]
        rs = sems[0:NB]
        cs = sems[NB:2 * NB]
        gs = sems[2 * NB:3 * NB]
        ss = sems[3 * NB:4 * NB]
        sem_t = sems[4 * NB]
        c = lax.axis_index("c")
        s = lax.axis_index("s")
        ro = jnp.minimum(s * strip, n_nodes - strip)
        base0 = s * ept
        rbase0 = c * e + base0

        def start_row_idx(k, b):
            pltpu.async_copy(row_hbm.at[pl.ds(rbase0 + k * K, K)], ridx[b], rs[b])

        def start_col_idx(k, b):
            pltpu.async_copy(col_hbm.at[pl.ds(base0 + k * K, K)], cidx[b], cs[b])

        def start_gather(b):
            pltpu.make_async_copy(row_hbm.at[pl.ds(rbase0, K)], ridx[b], rs[b]).wait()
            pltpu.async_copy(h_hbm.at[ridx[b]], rows[b], gs[b])

        def start_scatter(b):
            pltpu.make_async_copy(h_hbm.at[ridx[b]], rows[b], gs[b]).wait()
            pltpu.make_async_copy(col_hbm.at[pl.ds(base0, K)], cidx[b], cs[b]).wait()
            pltpu.async_copy(rows[b], acc.at[cidx[b]], ss[b], add=True)

        def wait_scatter(b):
            pltpu.make_async_copy(rows[b], acc.at[cidx[b]], ss[b]).wait()

        for b in range(NB):
            start_row_idx(b, b)
            start_col_idx(b, b)
        start_gather(0)
        pltpu.sync_copy(z_hbm.at[pl.ds(ro, strip)], acc.at[pl.ds(ro, strip)])
        for b in range(1, NB):
            start_gather(b)
        plsc.subcore_barrier()

        def group(kg, carry):
            k = NB * kg
            for b in range(NB):
                start_scatter(b)
                start_row_idx(k + NB + b, b)
            for b in range(NB):
                wait_scatter(b)
                start_col_idx(k + NB + b, b)
                start_gather(b)
            return carry

        lax.fori_loop(0, nch // NB - 1, group, 0)
        for b in range(NB):
            start_scatter(b)
        if tail:
            b = base0 + nch * K
            pltpu.sync_copy(row_hbm.at[pl.ds(rbase0 + nch * K, tail)], ridx_t)
            pltpu.async_copy(h_hbm.at[ridx_t], rows_t, sem_t).wait()
            pltpu.sync_copy(col_hbm.at[pl.ds(b, tail)], cidx_t)
            pltpu.async_copy(rows_t, acc.at[cidx_t], sem_t, add=True)
            pltpu.make_async_copy(rows_t, acc.at[cidx_t], sem_t).wait()
        for b in range(NB):
            wait_scatter(b)
        plsc.subcore_barrier()
        pltpu.sync_copy(acc.at[pl.ds(ro, strip)],
                        out_hbm.at[pl.ds(c * n_nodes + ro, strip)])

    zeros = jnp.zeros((n_nodes, 128), jnp.float32)
    return agg_kernel(rowcat, col, hp2, zeros)


def _mm1_body(x_ref, w_ref, dp_ref, hp_ref, dis_ref):
    dp = dp_ref[0, :, 0:1] + dp_ref[1, :, 0:1] + 1.0
    dis = lax.rsqrt(dp)
    h = jnp.dot(x_ref[...], w_ref[...], preferred_element_type=jnp.float32)
    hp = h * dis
    hp_ref[0] = hp[:, :128]
    hp_ref[1] = hp[:, 128:]
    dis_ref[...] = dis


def _tc_mm1(x, w1, degpad, n_nodes):
    f_in = x.shape[1]
    return pl.pallas_call(
        _mm1_body,
        grid=(n_nodes // BN,),
        in_specs=[
            pl.BlockSpec((BN, f_in), lambda i: (i, 0)),
            pl.BlockSpec((f_in, 256), lambda i: (0, 0)),
            pl.BlockSpec((2, BN, 128), lambda i: (0, i, 0)),
        ],
        out_specs=[
            pl.BlockSpec((2, BN, 128), lambda i: (0, i, 0)),
            pl.BlockSpec((BN, 1), lambda i: (i, 0)),
        ],
        out_shape=[
            jax.ShapeDtypeStruct((2, n_nodes, 128), jnp.float32),
            jax.ShapeDtypeStruct((n_nodes, 1), jnp.float32),
        ],
    )(x, w1, degpad)


def _actmm_body(s_ref, hp_ref, dis_ref, b_ref, w_ref, o_ref):
    dis = dis_ref[...]
    t = jnp.concatenate([s_ref[0] + hp_ref[0], s_ref[1] + hp_ref[1]], axis=1)
    pre = t * dis + b_ref[...]
    a = jnp.where(pre >= 0, pre, NEG * pre)
    h = jnp.dot(a, w_ref[...], preferred_element_type=jnp.float32)
    hp = h * dis
    o_ref[0] = hp[:, :128]
    o_ref[1] = hp[:, 128:]


def _tc_actmm(s3, hp, dis, b, w, n_nodes):
    return pl.pallas_call(
        _actmm_body,
        grid=(n_nodes // BN,),
        in_specs=[
            pl.BlockSpec((2, BN, 128), lambda i: (0, i, 0)),
            pl.BlockSpec((2, BN, 128), lambda i: (0, i, 0)),
            pl.BlockSpec((BN, 1), lambda i: (i, 0)),
            pl.BlockSpec((1, 256), lambda i: (0, 0)),
            pl.BlockSpec((256, 256), lambda i: (0, 0)),
        ],
        out_specs=pl.BlockSpec((2, BN, 128), lambda i: (0, i, 0)),
        out_shape=jax.ShapeDtypeStruct((2, n_nodes, 128), jnp.float32),
    )(s3, hp, dis, b, w)


def _final_body(s_ref, hp_ref, dis_ref, b_ref, bt_ref,
                fw1_ref, fb1_ref, fw2_ref, fb2_ref, o_ref, sums, cnts):
    i = pl.program_id(0)

    @pl.when(i == 0)
    def _init():
        sums[...] = jnp.zeros_like(sums)
        cnts[...] = jnp.zeros_like(cnts)

    dis = dis_ref[...]
    t = jnp.concatenate([s_ref[0] + hp_ref[0], s_ref[1] + hp_ref[1]], axis=1)
    out3 = t * dis + b_ref[...]
    gi = lax.broadcasted_iota(jnp.int32, (BN, 64), 1)
    oh = (gi == bt_ref[...]).astype(jnp.float32)
    sums[...] += lax.dot_general(oh, out3, (((0,), (0,)), ((), ())),
                                 preferred_element_type=jnp.float32)
    cnts[...] += lax.dot_general(oh, jnp.ones((BN, 128), jnp.float32),
                                 (((0,), (0,)), ((), ())),
                                 preferred_element_type=jnp.float32)

    @pl.when(i == pl.num_programs(0) - 1)
    def _fin():
        cnt = jnp.concatenate([cnts[...], cnts[...]], axis=1)
        emb = sums[...] / jnp.maximum(cnt, 1.0)
        e1 = jnp.dot(emb, fw1_ref[...], preferred_element_type=jnp.float32)
        e1 = e1 + fb1_ref[...]
        e1 = jnp.where(e1 >= 0, e1, NEG * e1)
        out = jnp.dot(e1, fw2_ref[...], preferred_element_type=jnp.float32)
        o_ref[...] = out + fb2_ref[...]


def _tc_final(s3, hp, dis, b, batch2, fw1, fb1, fw2, fb2, n_nodes, n_cls):
    return pl.pallas_call(
        _final_body,
        grid=(n_nodes // BN,),
        in_specs=[
            pl.BlockSpec((2, BN, 128), lambda i: (0, i, 0)),
            pl.BlockSpec((2, BN, 128), lambda i: (0, i, 0)),
            pl.BlockSpec((BN, 1), lambda i: (i, 0)),
            pl.BlockSpec((1, 256), lambda i: (0, 0)),
            pl.BlockSpec((BN, 1), lambda i: (i, 0)),
            pl.BlockSpec((256, 256), lambda i: (0, 0)),
            pl.BlockSpec((1, 256), lambda i: (0, 0)),
            pl.BlockSpec((256, n_cls), lambda i: (0, 0)),
            pl.BlockSpec((1, n_cls), lambda i: (0, 0)),
        ],
        out_specs=pl.BlockSpec((64, n_cls), lambda i: (0, 0)),
        out_shape=jax.ShapeDtypeStruct((64, n_cls), jnp.float32),
        scratch_shapes=[
            pltpu.VMEM((64, 256), jnp.float32),
            pltpu.VMEM((64, 128), jnp.float32),
        ],
        compiler_params=pltpu.CompilerParams(
            dimension_semantics=("arbitrary",)),
    )(s3, hp, dis, b, batch2, fw1, fb1, fw2, fb2)


def kernel(x, edge_index, batch, W1, b1, W2, b2, W3, b3, FW1, Fb1, FW2, Fb2):
    n = x.shape[0]
    row = edge_index[0]
    col = edge_index[1]
    rowcat = jnp.concatenate([row, row + n])
    batch2 = batch.reshape(n, 1)
    b1r = b1.reshape(1, -1)
    b2r = b2.reshape(1, -1)
    b3r = b3.reshape(1, -1)
    fb1r = Fb1.reshape(1, -1)
    fb2r = Fb2.reshape(1, -1)

    degpad = _sc_degree(col, n).reshape(2, n, 128)
    hp1, dis = _tc_mm1(x, W1, degpad, n)
    s1 = _sc_aggregate(hp1.reshape(2 * n, 128), rowcat, col, n).reshape(2, n, 128)
    hp2 = _tc_actmm(s1, hp1, dis, b1r, W2, n)
    s2 = _sc_aggregate(hp2.reshape(2 * n, 128), rowcat, col, n).reshape(2, n, 128)
    hp3 = _tc_actmm(s2, hp2, dis, b2r, W3, n)
    s3 = _sc_aggregate(hp3.reshape(2 * n, 128), rowcat, col, n).reshape(2, n, 128)
    return _tc_final(s3, hp3, dis, b3r, batch2, FW1, fb1r, FW2, fb2r,
                     n, FW2.shape[1])


# speedup vs baseline: 18.6836x; 1.2058x over previous
import functools

import jax
import jax.numpy as jnp
from jax import lax
from jax.experimental import pallas as pl
from jax.experimental.pallas import tpu as pltpu
from jax.experimental.pallas import tpu_sc as plsc

NEG = 0.01
NC = 2
NS = 16
BN = 400
K = 128


def _sc_degree(col, n_nodes):
    e = col.shape[0]
    epc = e // NC
    ept = epc // NS
    nch = ept // K
    tail = ept - nch * K
    strip = (-(-n_nodes // NS) + 7) // 8 * 8
    mesh = plsc.VectorSubcoreMesh(core_axis_name="c", subcore_axis_name="s",
                                  num_cores=NC, num_subcores=NS)

    assert nch % 2 == 0 and nch >= 4

    @functools.partial(
        pl.kernel,
        out_type=jax.ShapeDtypeStruct((NC * n_nodes, 128), jnp.float32),
        mesh=mesh,
        scratch_types=[
            pltpu.VMEM((K,), jnp.int32), pltpu.VMEM((K,), jnp.int32),
            pltpu.VMEM((tail,), jnp.int32),
            pltpu.VMEM((K, 128), jnp.float32),
            pltpu.MemorySpace.VMEM_SHARED((n_nodes, 128), jnp.float32),
            pltpu.SemaphoreType.DMA, pltpu.SemaphoreType.DMA,
            pltpu.SemaphoreType.DMA, pltpu.SemaphoreType.DMA,
            pltpu.SemaphoreType.DMA,
        ],
    )
    def deg_kernel(col_hbm, ones_hbm, z_hbm, out_hbm,
                   cidx0, cidx1, cidx_t, ones_v, acc,
                   cs0, cs1, ss0, ss1, sem_t):
        c = lax.axis_index("c")
        s = lax.axis_index("s")
        cidx = (cidx0, cidx1)
        cs = (cs0, cs1)
        ss = (ss0, ss1)
        ro = jnp.minimum(s * strip, n_nodes - strip)
        base0 = c * epc + s * ept

        def start_idx(k, b):
            pltpu.async_copy(col_hbm.at[pl.ds(base0 + k * K, K)], cidx[b], cs[b])

        def start_scatter(b):
            pltpu.make_async_copy(col_hbm.at[pl.ds(base0, K)], cidx[b], cs[b]).wait()
            pltpu.async_copy(ones_v, acc.at[cidx[b]], ss[b], add=True)

        def wait_scatter(b):
            pltpu.make_async_copy(ones_v, acc.at[cidx[b]], ss[b]).wait()

        start_idx(0, 0)
        start_idx(1, 1)
        pltpu.sync_copy(z_hbm.at[pl.ds(ro, strip)], acc.at[pl.ds(ro, strip)])
        pltpu.sync_copy(ones_hbm, ones_v)
        plsc.subcore_barrier()

        def pair(k2, carry):
            k = 2 * k2
            start_scatter(0)
            start_scatter(1)
            wait_scatter(0)
            start_idx(k + 2, 0)
            wait_scatter(1)
            start_idx(k + 3, 1)
            return carry

        lax.fori_loop(0, nch // 2 - 1, pair, 0)
        start_scatter(0)
        start_scatter(1)
        if tail:
            b = base0 + nch * K
            pltpu.sync_copy(col_hbm.at[pl.ds(b, tail)], cidx_t)
            pltpu.async_copy(ones_v.at[pl.ds(0, tail)], acc.at[cidx_t], sem_t,
                             add=True)
            pltpu.make_async_copy(ones_v.at[pl.ds(0, tail)], acc.at[cidx_t],
                                  sem_t).wait()
        wait_scatter(0)
        wait_scatter(1)
        plsc.subcore_barrier()
        pltpu.sync_copy(acc.at[pl.ds(ro, strip)],
                        out_hbm.at[pl.ds(c * n_nodes + ro, strip)])

    ones = jnp.ones((K, 128), jnp.float32)
    zeros = jnp.zeros((n_nodes, 128), jnp.float32)
    return deg_kernel(col, ones, zeros)


def _sc_aggregate(hp2, rowcat, col, n_nodes):
    KA = 64
    e = rowcat.shape[0] // 2
    ept = e // NS
    nch = ept // KA
    tail = ept - nch * KA
    strip = (-(-n_nodes // NS) + 7) // 8 * 8
    mesh = plsc.VectorSubcoreMesh(core_axis_name="c", subcore_axis_name="s",
                                  num_cores=NC, num_subcores=NS)

    NB = 4
    assert nch % NB == 0 and nch >= 2 * NB

    @functools.partial(
        pl.kernel,
        out_type=jax.ShapeDtypeStruct((NC * n_nodes, 128), jnp.float32),
        mesh=mesh,
        scratch_types=(
            [pltpu.VMEM((KA,), jnp.int32)] * NB
            + [pltpu.VMEM((KA,), jnp.int32)] * NB
            + [pltpu.VMEM((KA, 128), jnp.float32)] * NB
            + [pltpu.VMEM((tail,), jnp.int32),
               pltpu.VMEM((tail,), jnp.int32),
               pltpu.VMEM((tail, 128), jnp.float32),
               pltpu.MemorySpace.VMEM_SHARED((n_nodes, 128), jnp.float32)]
            + [pltpu.SemaphoreType.DMA] * (4 * NB + 1)
        ),
    )
    def agg_kernel(row_hbm, col_hbm, h_hbm, z_hbm, out_hbm, *bufs):
        ridx = bufs[0:NB]
        cidx = bufs[NB:2 * NB]
        rows = bufs[2 * NB:3 * NB]
        ridx_t, cidx_t, rows_t, acc = bufs[3 * NB:3 * NB + 4]
        sems = bufs[3 * NB + 4:]
        rs = sems[0:NB]
        cs = sems[NB:2 * NB]
        gs = sems[2 * NB:3 * NB]
        ss = sems[3 * NB:4 * NB]
        sem_t = sems[4 * NB]
        c = lax.axis_index("c")
        s = lax.axis_index("s")
        ro = jnp.minimum(s * strip, n_nodes - strip)
        base0 = s * ept
        rbase0 = c * e + base0

        def start_row_idx(k, b):
            pltpu.async_copy(row_hbm.at[pl.ds(rbase0 + k * KA, KA)], ridx[b], rs[b])

        def start_col_idx(k, b):
            pltpu.async_copy(col_hbm.at[pl.ds(base0 + k * KA, KA)], cidx[b], cs[b])

        def start_gather(b):
            pltpu.make_async_copy(row_hbm.at[pl.ds(rbase0, KA)], ridx[b], rs[b]).wait()
            pltpu.async_copy(h_hbm.at[ridx[b]], rows[b], gs[b])

        def start_scatter(b):
            pltpu.make_async_copy(h_hbm.at[ridx[b]], rows[b], gs[b]).wait()
            pltpu.make_async_copy(col_hbm.at[pl.ds(base0, KA)], cidx[b], cs[b]).wait()
            pltpu.async_copy(rows[b], acc.at[cidx[b]], ss[b], add=True)

        def wait_scatter(b):
            pltpu.make_async_copy(rows[b], acc.at[cidx[b]], ss[b]).wait()

        for b in range(NB):
            start_row_idx(b, b)
            start_col_idx(b, b)
        start_gather(0)
        pltpu.sync_copy(z_hbm.at[pl.ds(ro, strip)], acc.at[pl.ds(ro, strip)])
        for b in range(1, NB):
            start_gather(b)
        plsc.subcore_barrier()

        def group(kg, carry):
            k = NB * kg
            for b in range(NB):
                start_scatter(b)
                start_row_idx(k + NB + b, b)
            for b in range(NB):
                wait_scatter(b)
                start_col_idx(k + NB + b, b)
                start_gather(b)
            return carry

        lax.fori_loop(0, nch // NB - 1, group, 0)
        for b in range(NB):
            start_scatter(b)
        if tail:
            b = base0 + nch * KA
            pltpu.sync_copy(row_hbm.at[pl.ds(rbase0 + nch * KA, tail)], ridx_t)
            pltpu.async_copy(h_hbm.at[ridx_t], rows_t, sem_t).wait()
            pltpu.sync_copy(col_hbm.at[pl.ds(b, tail)], cidx_t)
            pltpu.async_copy(rows_t, acc.at[cidx_t], sem_t, add=True)
            pltpu.make_async_copy(rows_t, acc.at[cidx_t], sem_t).wait()
        for b in range(NB):
            wait_scatter(b)
        plsc.subcore_barrier()
        pltpu.sync_copy(acc.at[pl.ds(ro, strip)],
                        out_hbm.at[pl.ds(c * n_nodes + ro, strip)])

    zeros = jnp.zeros((n_nodes, 128), jnp.float32)
    return agg_kernel(rowcat, col, hp2, zeros)


def _mm1_body(x_ref, w_ref, dp_ref, hp_ref, dis_ref):
    dp = dp_ref[0, :, 0:1] + dp_ref[1, :, 0:1] + 1.0
    dis = lax.rsqrt(dp)
    h = jnp.dot(x_ref[...], w_ref[...], preferred_element_type=jnp.float32)
    hp = h * dis
    hp_ref[0] = hp[:, :128]
    hp_ref[1] = hp[:, 128:]
    dis_ref[...] = dis


def _tc_mm1(x, w1, degpad, n_nodes):
    f_in = x.shape[1]
    return pl.pallas_call(
        _mm1_body,
        grid=(n_nodes // BN,),
        in_specs=[
            pl.BlockSpec((BN, f_in), lambda i: (i, 0)),
            pl.BlockSpec((f_in, 256), lambda i: (0, 0)),
            pl.BlockSpec((2, BN, 128), lambda i: (0, i, 0)),
        ],
        out_specs=[
            pl.BlockSpec((2, BN, 128), lambda i: (0, i, 0)),
            pl.BlockSpec((BN, 1), lambda i: (i, 0)),
        ],
        out_shape=[
            jax.ShapeDtypeStruct((2, n_nodes, 128), jnp.float32),
            jax.ShapeDtypeStruct((n_nodes, 1), jnp.float32),
        ],
    )(x, w1, degpad)


def _actmm_body(s_ref, hp_ref, dis_ref, b_ref, w_ref, o_ref):
    dis = dis_ref[...]
    t = jnp.concatenate([s_ref[0] + hp_ref[0], s_ref[1] + hp_ref[1]], axis=1)
    pre = t * dis + b_ref[...]
    a = jnp.where(pre >= 0, pre, NEG * pre)
    h = jnp.dot(a, w_ref[...], preferred_element_type=jnp.float32)
    hp = h * dis
    o_ref[0] = hp[:, :128]
    o_ref[1] = hp[:, 128:]


def _tc_actmm(s3, hp, dis, b, w, n_nodes):
    return pl.pallas_call(
        _actmm_body,
        grid=(n_nodes // BN,),
        in_specs=[
            pl.BlockSpec((2, BN, 128), lambda i: (0, i, 0)),
            pl.BlockSpec((2, BN, 128), lambda i: (0, i, 0)),
            pl.BlockSpec((BN, 1), lambda i: (i, 0)),
            pl.BlockSpec((1, 256), lambda i: (0, 0)),
            pl.BlockSpec((256, 256), lambda i: (0, 0)),
        ],
        out_specs=pl.BlockSpec((2, BN, 128), lambda i: (0, i, 0)),
        out_shape=jax.ShapeDtypeStruct((2, n_nodes, 128), jnp.float32),
    )(s3, hp, dis, b, w)


def _final_body(s_ref, hp_ref, dis_ref, b_ref, bt_ref,
                fw1_ref, fb1_ref, fw2_ref, fb2_ref, o_ref, sums, cnts):
    i = pl.program_id(0)

    @pl.when(i == 0)
    def _init():
        sums[...] = jnp.zeros_like(sums)
        cnts[...] = jnp.zeros_like(cnts)

    dis = dis_ref[...]
    t = jnp.concatenate([s_ref[0] + hp_ref[0], s_ref[1] + hp_ref[1]], axis=1)
    out3 = t * dis + b_ref[...]
    gi = lax.broadcasted_iota(jnp.int32, (BN, 64), 1)
    oh = (gi == bt_ref[...]).astype(jnp.float32)
    sums[...] += lax.dot_general(oh, out3, (((0,), (0,)), ((), ())),
                                 preferred_element_type=jnp.float32)
    cnts[...] += lax.dot_general(oh, jnp.ones((BN, 128), jnp.float32),
                                 (((0,), (0,)), ((), ())),
                                 preferred_element_type=jnp.float32)

    @pl.when(i == pl.num_programs(0) - 1)
    def _fin():
        cnt = jnp.concatenate([cnts[...], cnts[...]], axis=1)
        emb = sums[...] / jnp.maximum(cnt, 1.0)
        e1 = jnp.dot(emb, fw1_ref[...], preferred_element_type=jnp.float32)
        e1 = e1 + fb1_ref[...]
        e1 = jnp.where(e1 >= 0, e1, NEG * e1)
        out = jnp.dot(e1, fw2_ref[...], preferred_element_type=jnp.float32)
        o_ref[...] = out + fb2_ref[...]


def _tc_final(s3, hp, dis, b, batch2, fw1, fb1, fw2, fb2, n_nodes, n_cls):
    return pl.pallas_call(
        _final_body,
        grid=(n_nodes // BN,),
        in_specs=[
            pl.BlockSpec((2, BN, 128), lambda i: (0, i, 0)),
            pl.BlockSpec((2, BN, 128), lambda i: (0, i, 0)),
            pl.BlockSpec((BN, 1), lambda i: (i, 0)),
            pl.BlockSpec((1, 256), lambda i: (0, 0)),
            pl.BlockSpec((BN, 1), lambda i: (i, 0)),
            pl.BlockSpec((256, 256), lambda i: (0, 0)),
            pl.BlockSpec((1, 256), lambda i: (0, 0)),
            pl.BlockSpec((256, n_cls), lambda i: (0, 0)),
            pl.BlockSpec((1, n_cls), lambda i: (0, 0)),
        ],
        out_specs=pl.BlockSpec((64, n_cls), lambda i: (0, 0)),
        out_shape=jax.ShapeDtypeStruct((64, n_cls), jnp.float32),
        scratch_shapes=[
            pltpu.VMEM((64, 256), jnp.float32),
            pltpu.VMEM((64, 128), jnp.float32),
        ],
        compiler_params=pltpu.CompilerParams(
            dimension_semantics=("arbitrary",)),
    )(s3, hp, dis, b, batch2, fw1, fb1, fw2, fb2)


def kernel(x, edge_index, batch, W1, b1, W2, b2, W3, b3, FW1, Fb1, FW2, Fb2):
    n = x.shape[0]
    row = edge_index[0]
    col = edge_index[1]
    rowcat = jnp.concatenate([row, row + n])
    batch2 = batch.reshape(n, 1)
    b1r = b1.reshape(1, -1)
    b2r = b2.reshape(1, -1)
    b3r = b3.reshape(1, -1)
    fb1r = Fb1.reshape(1, -1)
    fb2r = Fb2.reshape(1, -1)

    degpad = _sc_degree(col, n).reshape(2, n, 128)
    hp1, dis = _tc_mm1(x, W1, degpad, n)
    s1 = _sc_aggregate(hp1.reshape(2 * n, 128), rowcat, col, n).reshape(2, n, 128)
    hp2 = _tc_actmm(s1, hp1, dis, b1r, W2, n)
    s2 = _sc_aggregate(hp2.reshape(2 * n, 128), rowcat, col, n).reshape(2, n, 128)
    hp3 = _tc_actmm(s2, hp2, dis, b2r, W3, n)
    s3 = _sc_aggregate(hp3.reshape(2 * n, 128), rowcat, col, n).reshape(2, n, 128)
    return _tc_final(s3, hp3, dis, b3r, batch2, FW1, fb1r, FW2, fb2r,
                     n, FW2.shape[1])
